# baseline (speedup 1.0000x reference)
import functools

import jax
import jax.numpy as jnp
from jax import lax
from jax.experimental import pallas as pl
from jax.experimental.pallas import tpu as pltpu
from jax.experimental.pallas import tpu_sc as plsc

MAXLEN = 200
VOCAB = 100000
EMBED_DIM = 64
BATCH = 1024

NW = 32
NTOK = BATCH * MAXLEN
TPW = NTOK // NW
G = 80
NG = TPW // G
NPAIR = G // 2
RG = 4
RW = 8
NIT = NG // RG


def _make_kernel():
    mesh = plsc.VectorSubcoreMesh(core_axis_name="c", subcore_axis_name="s")

    @functools.partial(
        pl.kernel,
        out_type=jax.ShapeDtypeStruct((NTOK // 2, 128), jnp.float32),
        mesh=mesh,
        scratch_types=[
            pltpu.VMEM((TPW + 16,), jnp.int32),
            pltpu.VMEM((TPW,), jnp.int32),
            pltpu.VMEM((RG, G, 128), jnp.float32),
            pltpu.VMEM((RW, NPAIR, 128), jnp.float32),
            pltpu.VMEM((5, NPAIR, 128), jnp.float32),
            pltpu.SemaphoreType.DMA((RG,)),
            pltpu.SemaphoreType.DMA((RW,)),
        ],
    )
    def tok_pos_kernel(x_hbm, tok_hbm, pos_hbm, out_hbm,
                       idx_all, gidx_all, rows_v, wbuf, pos_v,
                       sem_g, sem_o):
        wid = lax.axis_index("s") * 2 + lax.axis_index("c")
        tok0 = wid * TPW
        grp0 = wid * NG

        pltpu.sync_copy(x_hbm.at[pl.ds(tok0, TPW)], idx_all.at[pl.ds(0, TPW)])
        pltpu.sync_copy(pos_hbm, pos_v)

        def shift_body(i, c):
            for v in range(5):
                sl = pl.ds(i * G + v * 16, 16)
                gidx_all[sl] = lax.shift_right_logical(idx_all[sl], 1)
            return c

        lax.fori_loop(0, NG, shift_body, 0)

        def fire_gather(gg, b):
            pltpu.async_copy(
                tok_hbm.at[gidx_all.at[pl.ds(gg * G, G)]],
                rows_v.at[b], sem_g.at[b],
            )

        def drain_gather(b):
            pltpu.make_async_copy(
                tok_hbm.at[pl.ds(0, G)], rows_v.at[b], sem_g.at[b]
            ).wait()

        def fire_out(gg, wb):
            row0 = (grp0 + gg) * NPAIR
            pltpu.async_copy(
                wbuf.at[wb], out_hbm.at[pl.ds(row0, NPAIR)], sem_o.at[wb]
            )

        def drain_out(wb):
            pltpu.make_async_copy(
                wbuf.at[wb], out_hbm.at[pl.ds(0, NPAIR)], sem_o.at[wb]
            ).wait()

        def add_group(gg, b, wb):
            pbi = lax.rem(gg, 5)

            def add_body(k, c):
                tA = gg * G + 2 * k
                tv = idx_all[pl.ds(tA, 16)]
                hA = (tv[0] & 1) * 64
                hB = (tv[1] & 1) * 64
                for j in range(4):
                    vA = rows_v[b, 2 * k, pl.ds(hA + 16 * j, 16)]
                    wbuf[wb, k, pl.ds(16 * j, 16)] = (
                        vA + pos_v[pbi, k, pl.ds(16 * j, 16)]
                    )
                for j in range(4):
                    vB = rows_v[b, 2 * k + 1, pl.ds(hB + 16 * j, 16)]
                    wbuf[wb, k, pl.ds(64 + 16 * j, 16)] = (
                        vB + pos_v[pbi, k, pl.ds(64 + 16 * j, 16)]
                    )
                return c

            lax.fori_loop(0, NPAIR, add_body, 0)

        for b in range(RG):
            fire_gather(b, b)

        for gg in range(2 * RG):
            b = gg % RG
            drain_gather(b)
            add_group(gg, b, gg % RW)
            fire_out(gg, gg % RW)
            fire_gather(gg + RG, b)

        def it_body(it, c):
            for b in range(RG):
                gg = it * RG + b
                drain_gather(b)
                drain_out(gg % RW)
                add_group(gg, b, gg % RW)
                fire_out(gg, gg % RW)
                fire_gather(gg + RG, b)
            return c

        lax.fori_loop(2, NIT - 1, it_body, 0)

        for b in range(RG):
            gg = NG - RG + b
            drain_gather(b)
            drain_out(gg % RW)
            add_group(gg, b, gg % RW)
            fire_out(gg, gg % RW)

        for wb in range(RW):
            drain_out(wb)

    return tok_pos_kernel


_kernel = _make_kernel()


@jax.jit
def kernel(x, token_table, pos_table):
    x_flat = x.astype(jnp.int32).reshape(NTOK)
    tok2 = token_table.reshape(VOCAB // 2, 128)
    pos2 = pos_table.reshape(MAXLEN // 2, 128)
    pos_ext = jnp.concatenate([pos2, pos2[:20]], axis=0)
    pos_blocks = jnp.stack(
        [lax.dynamic_slice_in_dim(pos_ext, pb, NPAIR) for pb in (0, 40, 80, 20, 60)]
    )
    out = _kernel(x_flat, tok2, pos_blocks)
    return out.reshape(BATCH, MAXLEN, EMBED_DIM)


# speedup vs baseline: 1.2150x; 1.2150x over previous
import functools

import jax
import jax.numpy as jnp
from jax import lax
from jax.experimental import pallas as pl
from jax.experimental.pallas import tpu as pltpu
from jax.experimental.pallas import tpu_sc as plsc

MAXLEN = 200
VOCAB = 100000
EMBED_DIM = 64
BATCH = 1024

NW = 32
NTOK = BATCH * MAXLEN
TPW = NTOK // NW
G = 80
NG = TPW // G
NPAIR = G // 2
R = 4
NIT = NG // R


def _make_kernel():
    mesh = plsc.VectorSubcoreMesh(core_axis_name="c", subcore_axis_name="s")

    @functools.partial(
        pl.kernel,
        out_type=jax.ShapeDtypeStruct((NTOK, EMBED_DIM), jnp.float32),
        mesh=mesh,
        scratch_types=[
            pltpu.VMEM((TPW,), jnp.int32),
            pltpu.VMEM((TPW,), jnp.float32),
            pltpu.VMEM((R, G, 128), jnp.float32),
            pltpu.VMEM((R, G, EMBED_DIM), jnp.float32),
            pltpu.VMEM((5, NPAIR, 128), jnp.float32),
            pltpu.SemaphoreType.DMA((R,)),
            pltpu.SemaphoreType.DMA((R,)),
        ],
    )
    def tok_pos_kernel(x_hbm, tok_hbm, pos_hbm, out_hbm,
                       gidx_all, h_all, rows_v, wbuf, pos_v,
                       sem_g, sem_o):
        wid = lax.axis_index("s") * 2 + lax.axis_index("c")
        tok0 = wid * TPW

        pltpu.sync_copy(pos_hbm, pos_v)
        pltpu.sync_copy(x_hbm.at[pl.ds(tok0, TPW)], gidx_all)

        def split_body(i, c):
            for v in range(5):
                sl = pl.ds(i * G + v * 16, 16)
                t = gidx_all[sl]
                h_all[sl] = (t & 1).astype(jnp.float32)
                gidx_all[sl] = lax.shift_right_logical(t, 1)
            return c

        lax.fori_loop(0, NG, split_body, 0)

        def fire_gather(gg, b):
            pltpu.async_copy(
                tok_hbm.at[gidx_all.at[pl.ds(gg * G, G)]],
                rows_v.at[b], sem_g.at[b],
            )

        def drain_gather(b):
            pltpu.make_async_copy(
                tok_hbm.at[pl.ds(0, G)], rows_v.at[b], sem_g.at[b]
            ).wait()

        def fire_out(gg, b):
            pltpu.async_copy(
                wbuf.at[b], out_hbm.at[pl.ds(tok0 + gg * G, G)], sem_o.at[b]
            )

        def drain_out(b):
            pltpu.make_async_copy(
                wbuf.at[b], out_hbm.at[pl.ds(0, G)], sem_o.at[b]
            ).wait()

        splat_dn = lax.GatherDimensionNumbers(
            offset_dims=(), collapsed_slice_dims=(0,), start_index_map=(0,)
        )

        def splat_lane(hv, lane):
            idx = (jnp.zeros((16,), jnp.int32) + lane)[:, None]
            return lax.gather(
                hv, idx, splat_dn, slice_sizes=(1,),
                mode=lax.GatherScatterMode.PROMISE_IN_BOUNDS,
            )

        def add_group(gg, b):
            pbi = lax.rem(gg, 5)

            def add_body(k, c):
                t = 2 * k
                wbase = t & ~15
                hv = h_all[pl.ds(gg * G + wbase, 16)]
                lane = t - wbase
                mA = splat_lane(hv, lane)
                mB = splat_lane(hv, lane + 1)
                for j in range(4):
                    sl = pl.ds(16 * j, 16)
                    sh = pl.ds(64 + 16 * j, 16)
                    lo = rows_v[b, t, sl]
                    hi = rows_v[b, t, sh]
                    wbuf[b, t, sl] = lo + mA * (hi - lo) + pos_v[pbi, k, sl]
                for j in range(4):
                    sl = pl.ds(16 * j, 16)
                    sh = pl.ds(64 + 16 * j, 16)
                    lo = rows_v[b, t + 1, sl]
                    hi = rows_v[b, t + 1, sh]
                    wbuf[b, t + 1, sl] = lo + mB * (hi - lo) + pos_v[pbi, k, sh]
                return c

            lax.fori_loop(0, NPAIR, add_body, 0)

        for b in range(R):
            fire_gather(b, b)

        for gg in range(R):
            b = gg % R
            drain_gather(b)
            add_group(gg, b)
            fire_out(gg, b)
            fire_gather(gg + R, b)

        def it_body(it, c):
            for b in range(R):
                gg = it * R + b
                drain_gather(b)
                drain_out(b)
                add_group(gg, b)
                fire_out(gg, b)
                fire_gather(gg + R, b)
            return c

        lax.fori_loop(1, NIT - 1, it_body, 0)

        for b in range(R):
            gg = NG - R + b
            drain_gather(b)
            drain_out(b)
            add_group(gg, b)
            fire_out(gg, b)

        for b in range(R):
            drain_out(b)

    return tok_pos_kernel


_kernel = _make_kernel()


@jax.jit
def kernel(x, token_table, pos_table):
    x_flat = x.astype(jnp.int32).reshape(NTOK)
    tok2 = token_table.reshape(VOCAB // 2, 128)
    pos2 = pos_table.reshape(MAXLEN // 2, 128)
    pos_ext = jnp.concatenate([pos2, pos2[:20]], axis=0)
    pos_blocks = jnp.stack(
        [lax.dynamic_slice_in_dim(pos_ext, pb, NPAIR) for pb in (0, 40, 80, 20, 60)]
    )
    out = _kernel(x_flat, tok2, pos_blocks)
    return out.reshape(BATCH, MAXLEN, EMBED_DIM)


# speedup vs baseline: 1.8386x; 1.5133x over previous
import functools

import jax
import jax.numpy as jnp
from jax import lax
from jax.experimental import pallas as pl
from jax.experimental.pallas import tpu as pltpu
from jax.experimental.pallas import tpu_sc as plsc

MAXLEN = 200
VOCAB = 100000
EMBED_DIM = 64
BATCH = 1024

NW = 32
NTOK = BATCH * MAXLEN
TPW = NTOK // NW
G = 80
NG = TPW // G
NPAIR = G // 2
R = 4
NIT = NG // R


def _make_kernel():
    mesh = plsc.VectorSubcoreMesh(core_axis_name="c", subcore_axis_name="s")

    @functools.partial(
        pl.kernel,
        out_type=jax.ShapeDtypeStruct((NTOK, EMBED_DIM), jnp.float32),
        mesh=mesh,
        scratch_types=[
            pltpu.VMEM((TPW,), jnp.int32),
            pltpu.VMEM((TPW,), jnp.float32),
            pltpu.VMEM((R, G, 128), jnp.float32),
            pltpu.VMEM((R, G, EMBED_DIM), jnp.float32),
            pltpu.VMEM((5, NPAIR, 128), jnp.float32),
            pltpu.SemaphoreType.DMA((R,)),
            pltpu.SemaphoreType.DMA((R,)),
        ],
    )
    def tok_pos_kernel(x_hbm, tok_hbm, pos_hbm, out_hbm,
                       gidx_all, h_all, rows_v, wbuf, pos_v,
                       sem_g, sem_o):
        wid = lax.axis_index("s") * 2 + lax.axis_index("c")
        tok0 = wid * TPW

        pltpu.sync_copy(pos_hbm, pos_v)
        pltpu.sync_copy(x_hbm.at[pl.ds(tok0, TPW)], gidx_all)

        def split_body(i, c):
            for v in range(5):
                sl = pl.ds(i * G + v * 16, 16)
                t = gidx_all[sl]
                h_all[sl] = (t & 1).astype(jnp.float32)
                gidx_all[sl] = lax.shift_right_logical(t, 1)
            return c

        lax.fori_loop(0, NG, split_body, 0)

        def fire_gather(gg, b):
            pltpu.async_copy(
                tok_hbm.at[gidx_all.at[pl.ds(gg * G, G)]],
                rows_v.at[b], sem_g.at[b],
            )

        def drain_gather(b):
            pltpu.make_async_copy(
                tok_hbm.at[pl.ds(0, G)], rows_v.at[b], sem_g.at[b]
            ).wait()

        def fire_out(gg, b):
            pltpu.async_copy(
                wbuf.at[b], out_hbm.at[pl.ds(tok0 + gg * G, G)], sem_o.at[b]
            )

        def drain_out(b):
            pltpu.make_async_copy(
                wbuf.at[b], out_hbm.at[pl.ds(0, G)], sem_o.at[b]
            ).wait()

        splat_dn = lax.GatherDimensionNumbers(
            offset_dims=(), collapsed_slice_dims=(0,), start_index_map=(0,)
        )

        def splat_lane(hv, lane):
            idx = (jnp.zeros((16,), jnp.int32) + lane)[:, None]
            return lax.gather(
                hv, idx, splat_dn, slice_sizes=(1,),
                mode=lax.GatherScatterMode.PROMISE_IN_BOUNDS,
            )

        def add_group(gg, b):
            pbi = lax.rem(gg, 5)

            @plsc.parallel_loop(0, NPAIR, unroll=2)
            def add_body(k):
                t = 2 * k
                wbase = t & ~15
                hv = h_all[pl.ds(gg * G + wbase, 16)]
                lane = t - wbase
                mA = splat_lane(hv, lane)
                mB = splat_lane(hv, lane + 1)
                for j in range(4):
                    sl = pl.ds(16 * j, 16)
                    sh = pl.ds(64 + 16 * j, 16)
                    lo = rows_v[b, t, sl]
                    hi = rows_v[b, t, sh]
                    wbuf[b, t, sl] = lo + mA * (hi - lo) + pos_v[pbi, k, sl]
                for j in range(4):
                    sl = pl.ds(16 * j, 16)
                    sh = pl.ds(64 + 16 * j, 16)
                    lo = rows_v[b, t + 1, sl]
                    hi = rows_v[b, t + 1, sh]
                    wbuf[b, t + 1, sl] = lo + mB * (hi - lo) + pos_v[pbi, k, sh]

        for b in range(R):
            fire_gather(b, b)

        for gg in range(R):
            b = gg % R
            drain_gather(b)
            add_group(gg, b)
            fire_out(gg, b)
            fire_gather(gg + R, b)

        def it_body(it, c):
            for b in range(R):
                gg = it * R + b
                drain_gather(b)
                drain_out(b)
                add_group(gg, b)
                fire_out(gg, b)
                fire_gather(gg + R, b)
            return c

        lax.fori_loop(1, NIT - 1, it_body, 0)

        for b in range(R):
            gg = NG - R + b
            drain_gather(b)
            drain_out(b)
            add_group(gg, b)
            fire_out(gg, b)

        for b in range(R):
            drain_out(b)

    return tok_pos_kernel


_kernel = _make_kernel()


@jax.jit
def kernel(x, token_table, pos_table):
    x_flat = x.astype(jnp.int32).reshape(NTOK)
    tok2 = token_table.reshape(VOCAB // 2, 128)
    pos2 = pos_table.reshape(MAXLEN // 2, 128)
    pos_ext = jnp.concatenate([pos2, pos2[:20]], axis=0)
    pos_blocks = jnp.stack(
        [lax.dynamic_slice_in_dim(pos_ext, pb, NPAIR) for pb in (0, 40, 80, 20, 60)]
    )
    out = _kernel(x_flat, tok2, pos_blocks)
    return out.reshape(BATCH, MAXLEN, EMBED_DIM)
